# baseline (speedup 1.0000x reference)
import functools

import jax
import jax.numpy as jnp
from jax import lax
from jax.experimental import pallas as pl
from jax.experimental.pallas import tpu as pltpu
from jax.experimental.pallas import tpu_sc as plsc

NC = 2
NS = 16
L = 16
NW = NC * NS

BATCH = 16384
D = 16
B_PER_W = BATCH // NW
S = 16
N_CHUNKS = B_PER_W // S


def _gmf_body(uidx_hbm, iidx_hbm, utab_hbm, itab_hbm, w_hbm, b_hbm,
              out_hbm, uidx_v, iidx_v, ublk_v, iblk_v,
              w_v, b_v, out_v, sem):
    wid = lax.axis_index("s") * NC + lax.axis_index("c")
    base = wid * B_PER_W

    pltpu.sync_copy(uidx_hbm.at[pl.ds(base, B_PER_W)], uidx_v)
    pltpu.sync_copy(iidx_hbm.at[pl.ds(base, B_PER_W)], iidx_v)
    pltpu.sync_copy(w_hbm, w_v)
    pltpu.sync_copy(b_hbm, b_v)

    lane = lax.iota(jnp.int32, L)
    b_vec = b_v[...]
    w_vec = w_v[...]

    def chunk(c, _):
        uvec = uidx_v[pl.ds(c * S, S)]
        ivec = iidx_v[pl.ds(c * S, S)]
        for j in range(S):
            cbu = pl.multiple_of((uvec[j] >> 7) * 128, 128)
            cbi = pl.multiple_of((ivec[j] >> 7) * 128, 128)
            pltpu.async_copy(utab_hbm.at[:, pl.ds(cbu, 128)],
                             ublk_v.at[:, pl.ds(j * 128, 128)], sem)
            pltpu.async_copy(itab_hbm.at[:, pl.ds(cbi, 128)],
                             iblk_v.at[:, pl.ds(j * 128, 128)], sem)
        pltpu.make_async_copy(utab_hbm.at[:, pl.ds(0, S * 128)], ublk_v,
                              sem).wait()
        pltpu.make_async_copy(utab_hbm.at[:, pl.ds(0, S * 128)], iblk_v,
                              sem).wait()

        res = jnp.zeros((L,), jnp.float32)
        for j in range(S):
            cu = jnp.full((L,), j * 128, jnp.int32) + (uvec[j] & 127)
            ci = jnp.full((L,), j * 128, jnp.int32) + (ivec[j] & 127)
            ucol = plsc.load_gather(ublk_v, [lane, cu])
            icol = plsc.load_gather(iblk_v, [lane, ci])
            s = jnp.sum(ucol * icol * w_vec)
            res = jnp.where(lane == j, res + s, res)
        logits = res + b_vec
        out_v[pl.ds(c * S, S)] = 1.0 / (1.0 + jnp.exp(-logits))
        return _

    lax.fori_loop(0, N_CHUNKS, chunk, None)

    pltpu.sync_copy(out_v, out_hbm.at[pl.ds(base, B_PER_W)])


@functools.partial(jax.jit, static_argnames=())
def kernel(user_input, item_input, user_table, item_table, W, b):
    mesh = plsc.VectorSubcoreMesh(
        core_axis_name="c", subcore_axis_name="s",
        num_cores=NC, num_subcores=NS)
    k = pl.kernel(
        _gmf_body,
        out_type=jax.ShapeDtypeStruct((BATCH,), jnp.float32),
        mesh=mesh,
        scratch_types=[
            pltpu.VMEM((B_PER_W,), jnp.int32),
            pltpu.VMEM((B_PER_W,), jnp.int32),
            pltpu.VMEM((D, S * 128), jnp.float32),
            pltpu.VMEM((D, S * 128), jnp.float32),
            pltpu.VMEM((D,), jnp.float32),
            pltpu.VMEM((L,), jnp.float32),
            pltpu.VMEM((B_PER_W,), jnp.float32),
            pltpu.SemaphoreType.DMA,
        ],
        compiler_params=pltpu.CompilerParams(
            needs_layout_passes=False, use_tc_tiling_on_sc=True),
        name="gmf_sc",
    )
    w16 = W.reshape(D)
    b16 = jnp.broadcast_to(b, (L,))
    return k(user_input, item_input, user_table.T, item_table.T, w16, b16)
